# baseline (speedup 1.0000x reference)
import functools

import jax
import jax.numpy as jnp
from jax import lax
from jax.experimental import pallas as pl
from jax.experimental.pallas import tpu as pltpu
from jax.experimental.pallas import tpu_sc as plsc

B = 1024
T = 100000
N = 8192
F = 16

L = 16
TT = T // 8
ROWS64 = B * T // L
NG = 64
CHN = 128
NCH = N // CHN


def _make_sc_kernel():
    info = plsc.get_sparse_core_info()
    nc, ns = info.num_cores, info.num_subcores
    nw = nc * ns
    gper = NG // nw

    mesh = plsc.VectorSubcoreMesh(core_axis_name="c", subcore_axis_name="s")

    @functools.partial(
        pl.kernel,
        mesh=mesh,
        out_type=(),
        compiler_params=pltpu.CompilerParams(
            needs_layout_passes=False, use_tc_tiling_on_sc=False),
        scratch_types=[
            pltpu.VMEM((2, F, CHN), jnp.int32),
            pltpu.VMEM((2, F, CHN), jnp.float32),
            pltpu.VMEM((2, F * CHN, L), jnp.float32),
            pltpu.VMEM((2, CHN), jnp.float32),
            pltpu.VMEM((2, CHN), jnp.int32),
            pltpu.VMEM((4, CHN), jnp.int32),
            pltpu.VMEM((4, CHN, L), jnp.float32),
            pltpu.SemaphoreType.DMA,
            pltpu.SemaphoreType.DMA,
            pltpu.SemaphoreType.DMA,
        ],
    )
    def k(tape64_hbm, idxg_hbm, wt_hbm, bias_hbm, og_hbm, out64_ref,
          idx_v, w_v, g_v, b_v, ob_v, o_ring, x_ring, isem, gsem, ssem):
        cid = lax.axis_index("c")
        sid = lax.axis_index("s")
        wid = sid * nc + cid

        def start_inputs(g, c, p):
            co = c * CHN
            pltpu.make_async_copy(
                idxg_hbm.at[g, :, pl.ds(co, CHN)], idx_v.at[p], isem).start()
            pltpu.make_async_copy(
                wt_hbm.at[:, pl.ds(co, CHN)], w_v.at[p], isem).start()
            pltpu.make_async_copy(
                bias_hbm.at[pl.ds(co, CHN)], b_v.at[p], isem).start()
            pltpu.make_async_copy(
                og_hbm.at[g, pl.ds(co, CHN)], ob_v.at[p], isem).start()

        def wait_inputs(g, c, p):
            co = c * CHN
            pltpu.make_async_copy(
                idxg_hbm.at[g, :, pl.ds(co, CHN)], idx_v.at[p], isem).wait()
            pltpu.make_async_copy(
                wt_hbm.at[:, pl.ds(co, CHN)], w_v.at[p], isem).wait()
            pltpu.make_async_copy(
                bias_hbm.at[pl.ds(co, CHN)], b_v.at[p], isem).wait()
            pltpu.make_async_copy(
                og_hbm.at[g, pl.ds(co, CHN)], ob_v.at[p], isem).wait()

        def start_gather(p):
            for f in range(F):
                pltpu.make_async_copy(
                    tape64_hbm.at[idx_v.at[p, f]],
                    g_v.at[p, pl.ds(f * CHN, CHN)], gsem).start()

        def wait_gather(p):
            for f in range(F):
                pltpu.make_async_copy(
                    tape64_hbm.at[idx_v.at[p, f]],
                    g_v.at[p, pl.ds(f * CHN, CHN)], gsem).wait()

        def compute_chunk(p, slot):
            gp = g_v.at[p]
            wp = w_v.at[p]
            bp = b_v.at[p]
            xp = x_ring.at[slot]
            op = o_ring.at[slot]

            def node_body(n, carry):
                acc = plsc.load_gather(bp, [jnp.full((L,), n, jnp.int32)])
                for f in range(F):
                    row = gp[f * CHN + n]
                    ws = plsc.load_gather(
                        wp, [jnp.full((L,), f, jnp.int32),
                             jnp.full((L,), n, jnp.int32)])
                    acc = acc + row * ws
                xp[n] = jnp.maximum(acc, 0.0)
                return carry

            lax.fori_loop(0, CHN, node_body, 0)
            for q in range(CHN // L):
                op[pl.ds(q * L, L)] = ob_v[p, pl.ds(q * L, L)]

        def start_scatter(slot):
            pltpu.make_async_copy(
                x_ring.at[slot], out64_ref.at[o_ring.at[slot]], ssem).start()

        def drain_scatter():
            pltpu.make_async_copy(
                x_ring.at[0], out64_ref.at[o_ring.at[0]], ssem).wait()

        def group_body(gi):
            g = wid * gper + gi
            start_inputs(g, 0, 0)
            wait_inputs(g, 0, 0)
            start_gather(0)
            start_inputs(g, 1, 1)
            wait_inputs(g, 1, 1)
            start_gather(1)

            def chunk_step(c, p):
                slot = lax.rem(c, 4)
                wait_gather(p)

                @pl.when(c >= 4)
                def _():
                    drain_scatter()

                compute_chunk(p, slot)
                start_scatter(slot)

                @pl.when(c + 2 < NCH)
                def _():
                    start_inputs(g, c + 2, p)
                    wait_inputs(g, c + 2, p)
                    start_gather(p)

            def pair_body(i, carry):
                chunk_step(2 * i, 0)
                chunk_step(2 * i + 1, 1)
                return carry

            lax.fori_loop(0, NCH // 2, pair_body, 0)
            for _ in range(4):
                drain_scatter()

        for gi in range(gper):
            group_body(gi)

    return k


_sc_kernel = _make_sc_kernel()


def _to64(a):
    return (a.T.reshape(TT, 8, 8, 8, L)
             .transpose(0, 2, 1, 3, 4)
             .reshape(ROWS64, L))


def _from64(a64):
    return (a64.reshape(TT, 8, 8, 8, L)
               .transpose(0, 2, 1, 3, 4)
               .reshape(T, B).T)


def kernel(tape, input_indices, output_indices, weights, bias):
    garange = jnp.arange(NG, dtype=jnp.int32)
    goff = (garange // 8) * 64 + (garange % 8)
    ibase = (input_indices >> 3) * 512 + (input_indices & 7) * 8
    idxg = ibase.T[None] + goff[:, None, None]
    obase = (output_indices >> 3) * 512 + (output_indices & 7) * 8
    og = obase[None] + goff[:, None]
    wt = weights.T

    tape64 = _to64(tape)
    out_ref = jax.new_ref(_to64(jnp.copy(tape)))
    _sc_kernel(tape64, idxg, wt, bias, og, out_ref)
    return _from64(out_ref[...])


# speedup vs baseline: 3.2125x; 3.2125x over previous
import functools

import jax
import jax.numpy as jnp
from jax import lax
from jax.experimental import pallas as pl
from jax.experimental.pallas import tpu as pltpu
from jax.experimental.pallas import tpu_sc as plsc

B = 1024
T = 100000
N = 8192
F = 16

L = 16
CQ = B // L
CHN = 16


def _make_sc_kernel():
    info = plsc.get_sparse_core_info()
    nc, ns = info.num_cores, info.num_subcores
    nw = nc * ns
    nper = N // nw
    nchunk = nper // CHN

    mesh = plsc.VectorSubcoreMesh(core_axis_name="c", subcore_axis_name="s")

    @functools.partial(
        pl.kernel,
        mesh=mesh,
        out_type=(),
        compiler_params=pltpu.CompilerParams(
            needs_layout_passes=False, use_tc_tiling_on_sc=False),
        scratch_types=[
            pltpu.VMEM((nper, F), jnp.int32),
            pltpu.VMEM((nper * F,), jnp.float32),
            pltpu.VMEM((nper,), jnp.float32),
            pltpu.VMEM((nchunk, CHN), jnp.int32),
            pltpu.VMEM((2, F, B), jnp.float32),
            pltpu.VMEM((2, CHN, B), jnp.float32),
            pltpu.SemaphoreType.DMA,
            pltpu.SemaphoreType.DMA,
            pltpu.SemaphoreType.DMA,
        ],
    )
    def k(tapet_hbm, idx_hbm, w_hbm, bias_hbm, oidx_hbm, out_ref,
          idx_v, w_v, b_v, oidx_v, g_v, x_v, isem, gsem, ssem):
        cid = lax.axis_index("c")
        sid = lax.axis_index("s")
        wid = sid * nc + cid
        n0 = wid * nper

        pltpu.make_async_copy(
            idx_hbm.at[pl.ds(n0, nper)], idx_v, isem).start()
        pltpu.make_async_copy(
            w_hbm.at[pl.ds(n0 * F, nper * F)], w_v, isem).start()
        pltpu.make_async_copy(
            bias_hbm.at[pl.ds(n0, nper)], b_v, isem).start()
        pltpu.make_async_copy(
            oidx_hbm.at[pl.ds(wid * nchunk, nchunk)],
            oidx_v, isem).start()
        pltpu.make_async_copy(
            idx_hbm.at[pl.ds(n0, nper)], idx_v, isem).wait()
        pltpu.make_async_copy(
            w_hbm.at[pl.ds(n0 * F, nper * F)], w_v, isem).wait()
        pltpu.make_async_copy(
            bias_hbm.at[pl.ds(n0, nper)], b_v, isem).wait()
        pltpu.make_async_copy(
            oidx_hbm.at[pl.ds(wid * nchunk, nchunk)],
            oidx_v, isem).wait()

        def start_gather(j):
            pltpu.make_async_copy(
                tapet_hbm.at[idx_v.at[j]],
                g_v.at[lax.rem(j, 2)], gsem).start()

        def wait_gather(j):
            pltpu.make_async_copy(
                tapet_hbm.at[idx_v.at[j]],
                g_v.at[lax.rem(j, 2)], gsem).wait()

        def start_scatter(c):
            p = lax.rem(c, 2)
            pltpu.make_async_copy(
                x_v.at[p], out_ref.at[oidx_v.at[c]], ssem).start()

        def drain_scatter():
            pltpu.make_async_copy(
                x_v.at[0], out_ref.at[oidx_v.at[0]], ssem).wait()

        start_gather(0)

        def node_body(j, carry):
            @pl.when(j + 1 < nper)
            def _():
                start_gather(j + 1)

            wait_gather(j)
            pg = lax.rem(j, 2)
            px = lax.rem(j // CHN, 2)
            kk = lax.rem(j, CHN)
            gp = g_v.at[pg]
            xp = x_v.at[px]

            wsp = [plsc.load_gather(
                w_v, [jnp.full((L,), j * F + f, jnp.int32)])
                for f in range(F)]
            bsp = plsc.load_gather(b_v, [jnp.full((L,), j, jnp.int32)])

            def col_body(q, carry2):
                lo = q * L
                acc = bsp
                for f in range(F):
                    acc = acc + gp[f, pl.ds(lo, L)] * wsp[f]
                xp[kk, pl.ds(lo, L)] = jnp.maximum(acc, 0.0)
                return carry2

            lax.fori_loop(0, CQ, col_body, 0)

            @pl.when(kk == CHN - 1)
            def _():
                c = j // CHN

                @pl.when(c >= 2)
                def _():
                    drain_scatter()

                start_scatter(c)

            return carry

        lax.fori_loop(0, nper, node_body, 0)
        drain_scatter()
        drain_scatter()

    return k


_sc_kernel = _make_sc_kernel()


def kernel(tape, input_indices, output_indices, weights, bias):
    tapet = tape.T
    wflat = weights.reshape(N * F)
    out = jnp.copy(tape)
    out_ref = jax.new_ref(out.T)
    _sc_kernel(tapet, input_indices, wflat, bias,
               output_indices.reshape(N // CHN, CHN), out_ref)
    return out_ref[...].T


# speedup vs baseline: 8.0409x; 2.5030x over previous
import functools

import jax
import jax.numpy as jnp
from jax import lax
from jax.experimental import pallas as pl
from jax.experimental.pallas import tpu as pltpu
from jax.experimental.pallas import tpu_sc as plsc

B = 1024
T = 100000
N = 8192
F = 16

L = 16
CH = 128
NCHUNK = N // CH
GROUPS = CH // L
FCH = F * CH
PACKED = FCH + CH
IDX_MASK = (1 << 17) - 1


def _make_sc_kernel():
    info = plsc.get_sparse_core_info()
    nc, ns = info.num_cores, info.num_subcores
    nw = nc * ns
    rows_per = B // nw

    mesh = plsc.VectorSubcoreMesh(core_axis_name="c", subcore_axis_name="s")

    @functools.partial(
        pl.kernel,
        mesh=mesh,
        out_type=jax.ShapeDtypeStruct((B, T), jnp.float32),
        compiler_params=pltpu.CompilerParams(needs_layout_passes=False),
        scratch_types=[
            pltpu.VMEM((T,), jnp.float32),
            pltpu.VMEM((PACKED,), jnp.int32),
            pltpu.VMEM((PACKED,), jnp.int32),
            pltpu.VMEM((N,), jnp.float32),
            pltpu.VMEM((N,), jnp.int32),
            pltpu.VMEM_SHARED((NCHUNK, PACKED), jnp.int32),
            pltpu.SemaphoreType.DMA,
            pltpu.SemaphoreType.DMA,
        ],
    )
    def k(tape_hbm, packed_hbm, oidx_hbm, out_hbm,
          row_v, buf0, buf1, x_v, oidx_v, packed_sp, sem0, sem1):
        cid = lax.axis_index("c")
        sid = lax.axis_index("s")
        wid = sid * nc + cid

        @pl.when(sid == 0)
        def _stage():
            pltpu.sync_copy(packed_hbm, packed_sp)

        pltpu.sync_copy(oidx_hbm, oidx_v)
        plsc.subcore_barrier()

        def compute_chunk(c, buf):
            def group_body(g, carry):
                lo = g * L
                acc = plsc.bitcast(buf[pl.ds(FCH + lo, L)], jnp.float32)
                for f in range(F):
                    word = buf[pl.ds(f * CH + lo, L)]
                    iv = word & IDX_MASK
                    wv = plsc.bitcast(word & ~IDX_MASK, jnp.float32)
                    vals = plsc.load_gather(row_v, [iv])
                    acc = acc + vals * wv
                x_v[pl.ds(c * CH + lo, L)] = jnp.maximum(acc, 0.0)
                return carry

            lax.fori_loop(0, GROUPS, group_body, 0, unroll=2)

        def row_body(j, carry):
            r = wid * rows_per + j
            pltpu.sync_copy(tape_hbm.at[r], row_v)
            pltpu.make_async_copy(packed_sp.at[0], buf0, sem0).start()

            def pair_body(i, carry2):
                c0 = 2 * i
                pltpu.make_async_copy(packed_sp.at[c0 + 1], buf1, sem1).start()
                pltpu.make_async_copy(packed_sp.at[c0], buf0, sem0).wait()
                compute_chunk(c0, buf0)

                @pl.when(c0 + 2 < NCHUNK)
                def _prefetch():
                    pltpu.make_async_copy(
                        packed_sp.at[c0 + 2], buf0, sem0).start()

                pltpu.make_async_copy(packed_sp.at[c0 + 1], buf1, sem1).wait()
                compute_chunk(c0 + 1, buf1)
                return carry2

            lax.fori_loop(0, NCHUNK // 2, pair_body, 0)

            def scat_body(g, carry2):
                lo = g * L
                oi = oidx_v[pl.ds(lo, L)]
                plsc.store_scatter(row_v, [oi], x_v[pl.ds(lo, L)])
                return carry2

            lax.fori_loop(0, N // L, scat_body, 0, unroll=4)
            pltpu.sync_copy(row_v, out_hbm.at[r])
            return carry

        lax.fori_loop(0, rows_per, row_body, 0)

    return k


_sc_kernel = _make_sc_kernel()


def kernel(tape, input_indices, output_indices, weights, bias):
    idx3 = input_indices.reshape(NCHUNK, CH, F).transpose(0, 2, 1)
    w3 = weights.reshape(NCHUNK, CH, F).transpose(0, 2, 1)
    wbits = lax.bitcast_convert_type(w3, jnp.int32)
    wtop = (wbits + (1 << 16)) & ~IDX_MASK
    idxw = idx3 | wtop
    b2i = lax.bitcast_convert_type(bias.reshape(NCHUNK, CH), jnp.int32)
    packed = jnp.concatenate([idxw.reshape(NCHUNK, FCH), b2i], axis=1)
    return _sc_kernel(tape, packed, output_indices)


# speedup vs baseline: 8.0966x; 1.0069x over previous
import functools

import jax
import jax.numpy as jnp
from jax import lax
from jax.experimental import pallas as pl
from jax.experimental.pallas import tpu as pltpu
from jax.experimental.pallas import tpu_sc as plsc

B = 1024
T = 100000
N = 8192
F = 16

L = 16
CH = 128
NCHUNK = N // CH
GROUPS = CH // L
FCH = F * CH
PACKED = FCH + CH
IDX_MASK = (1 << 17) - 1


def _make_sc_kernel():
    info = plsc.get_sparse_core_info()
    nc, ns = info.num_cores, info.num_subcores
    nw = nc * ns
    rows_per = B // nw

    mesh = plsc.VectorSubcoreMesh(core_axis_name="c", subcore_axis_name="s")

    @functools.partial(
        pl.kernel,
        mesh=mesh,
        out_type=jax.ShapeDtypeStruct((B, T), jnp.float32),
        compiler_params=pltpu.CompilerParams(needs_layout_passes=False),
        scratch_types=[
            pltpu.VMEM((T,), jnp.float32),
            pltpu.VMEM((PACKED,), jnp.int32),
            pltpu.VMEM((PACKED,), jnp.int32),
            pltpu.VMEM((N,), jnp.float32),
            pltpu.VMEM((N,), jnp.int32),
            pltpu.VMEM_SHARED((NCHUNK, PACKED), jnp.int32),
            pltpu.SemaphoreType.DMA,
            pltpu.SemaphoreType.DMA,
        ],
    )
    def k(tape_hbm, packed_hbm, oidx_hbm, out_hbm,
          row_v, buf0, buf1, x_v, oidx_v, packed_sp, sem0, sem1):
        cid = lax.axis_index("c")
        sid = lax.axis_index("s")
        wid = sid * nc + cid

        @pl.when(sid == 0)
        def _stage():
            pltpu.sync_copy(packed_hbm, packed_sp)

        pltpu.sync_copy(oidx_hbm, oidx_v)
        plsc.subcore_barrier()

        def compute_chunk(c, buf):
            def group_body(g, carry):
                lo = g * L
                acc = plsc.bitcast(buf[pl.ds(FCH + lo, L)], jnp.float32)
                for f in range(F):
                    word = buf[pl.ds(f * CH + lo, L)]
                    iv = word & IDX_MASK
                    wv = plsc.bitcast(word & ~IDX_MASK, jnp.float32)
                    vals = plsc.load_gather(row_v, [iv])
                    acc = acc + vals * wv
                x_v[pl.ds(c * CH + lo, L)] = jnp.maximum(acc, 0.0)
                return carry

            lax.fori_loop(0, GROUPS, group_body, 0)

        def row_body(j, carry):
            r = wid * rows_per + j
            pltpu.sync_copy(tape_hbm.at[r], row_v)
            pltpu.make_async_copy(packed_sp.at[0], buf0, sem0).start()

            def pair_body(i, carry2):
                c0 = 2 * i
                pltpu.make_async_copy(packed_sp.at[c0 + 1], buf1, sem1).start()
                pltpu.make_async_copy(packed_sp.at[c0], buf0, sem0).wait()
                compute_chunk(c0, buf0)

                @pl.when(c0 + 2 < NCHUNK)
                def _prefetch():
                    pltpu.make_async_copy(
                        packed_sp.at[c0 + 2], buf0, sem0).start()

                pltpu.make_async_copy(packed_sp.at[c0 + 1], buf1, sem1).wait()
                compute_chunk(c0 + 1, buf1)
                return carry2

            lax.fori_loop(0, NCHUNK // 2, pair_body, 0)

            def scat_body(g, carry2):
                lo = g * L
                oi = oidx_v[pl.ds(lo, L)]
                plsc.store_scatter(row_v, [oi], x_v[pl.ds(lo, L)])
                return carry2

            lax.fori_loop(0, N // L, scat_body, 0)
            pltpu.sync_copy(row_v, out_hbm.at[r])
            return carry

        lax.fori_loop(0, rows_per, row_body, 0)

    return k


_sc_kernel = _make_sc_kernel()


def kernel(tape, input_indices, output_indices, weights, bias):
    idx3 = input_indices.reshape(NCHUNK, CH, F).transpose(0, 2, 1)
    w3 = weights.reshape(NCHUNK, CH, F).transpose(0, 2, 1)
    wbits = lax.bitcast_convert_type(w3, jnp.int32)
    wtop = (wbits + (1 << 16)) & ~IDX_MASK
    idxw = idx3 | wtop
    b2i = lax.bitcast_convert_type(bias.reshape(NCHUNK, CH), jnp.int32)
    packed = jnp.concatenate([idxw.reshape(NCHUNK, FCH), b2i], axis=1)
    return _sc_kernel(tape, packed, output_indices)
